# baseline (speedup 1.0000x reference)
import functools

import jax
import jax.numpy as jnp
from jax import lax
from jax.experimental import pallas as pl
from jax.experimental.pallas import tpu as pltpu
from jax.experimental.pallas import tpu_sc as plsc

NC = 2
NS = 16
NW = NC * NS

D = 64
CHUNK = 512
SUB = 128
NSUB = CHUNK // SUB


@functools.partial(jax.jit, static_argnums=(2,))
def _emb_lookup(idx_flat, table, bpw):
    nch = bpw // CHUNK
    mesh = plsc.VectorSubcoreMesh(core_axis_name="c", subcore_axis_name="s")

    @functools.partial(
        pl.kernel,
        mesh=mesh,
        out_type=jax.ShapeDtypeStruct((idx_flat.shape[0], D), jnp.float32),
        scratch_types=[
            pltpu.VMEM((CHUNK,), jnp.int32),
            pltpu.VMEM((CHUNK, D), jnp.float32),
            pltpu.SemaphoreType.DMA,
        ],
        compiler_params=pltpu.CompilerParams(use_tc_tiling_on_sc=False),
    )
    def body(idx_hbm, w_hbm, out_hbm, idx_v, rows_v, sem):
        wid = lax.axis_index("s") * NC + lax.axis_index("c")
        base = wid * bpw

        def chunk_step(c, carry):
            off = base + c * CHUNK
            pltpu.sync_copy(idx_hbm.at[pl.ds(off, CHUNK)], idx_v)
            copies = [
                pltpu.async_copy(
                    w_hbm.at[idx_v.at[pl.ds(j * SUB, SUB)]],
                    rows_v.at[pl.ds(j * SUB, SUB)],
                    sem,
                )
                for j in range(NSUB)
            ]
            for cp in copies:
                cp.wait()
            pltpu.sync_copy(rows_v, out_hbm.at[pl.ds(off, CHUNK)])
            return carry

        lax.fori_loop(0, nch, chunk_step, 0)

    return body(idx_flat, table)


def kernel(input, W):
    idx = input.reshape(-1).astype(jnp.int32)
    bpw = idx.shape[0] // NW
    out = _emb_lookup(idx, W, bpw)
    return out.reshape(input.shape + (W.shape[1],))


# speedup vs baseline: 1.0238x; 1.0238x over previous
import functools

import jax
import jax.numpy as jnp
from jax import lax
from jax.experimental import pallas as pl
from jax.experimental.pallas import tpu as pltpu
from jax.experimental.pallas import tpu_sc as plsc

NC = 2
NS = 16
NW = NC * NS

D = 64
CHUNK = 512
SUB = 128
NSUB = CHUNK // SUB
NBUF = 2


@functools.partial(jax.jit, static_argnums=(2,))
def _emb_lookup(idx_flat, table, bpw):
    nch = bpw // CHUNK
    assert nch % NBUF == 0
    mesh = plsc.VectorSubcoreMesh(core_axis_name="c", subcore_axis_name="s")

    @functools.partial(
        pl.kernel,
        mesh=mesh,
        out_type=jax.ShapeDtypeStruct((idx_flat.shape[0], D), jnp.float32),
        scratch_types=[
            pltpu.VMEM((NBUF, CHUNK), jnp.int32),
            pltpu.VMEM((NBUF, CHUNK, D), jnp.float32),
        ]
        + [pltpu.SemaphoreType.DMA] * (3 * NBUF),
        compiler_params=pltpu.CompilerParams(use_tc_tiling_on_sc=False),
    )
    def body(idx_hbm, w_hbm, out_hbm, idx_v, rows_v, *sems):
        gsem = sems[0:NBUF]
        osem = sems[NBUF : 2 * NBUF]
        isem = sems[2 * NBUF : 3 * NBUF]
        wid = lax.axis_index("s") * NC + lax.axis_index("c")
        base = wid * bpw

        pltpu.async_copy(idx_hbm.at[pl.ds(base, CHUNK)], idx_v.at[0], isem[0])

        def step(g, carry):
            for b in range(NBUF):
                c = g * NBUF + b
                off = base + c * CHUNK
                pltpu.make_async_copy(
                    idx_hbm.at[pl.ds(off, CHUNK)], idx_v.at[b], isem[b]
                ).wait()
                @pl.when(c >= NBUF)
                def _():
                    pltpu.make_async_copy(
                        rows_v.at[b], out_hbm.at[pl.ds(off, CHUNK)], osem[b]
                    ).wait()
                copies = [
                    pltpu.async_copy(
                        w_hbm.at[idx_v.at[b].at[pl.ds(j * SUB, SUB)]],
                        rows_v.at[b].at[pl.ds(j * SUB, SUB)],
                        gsem[b],
                    )
                    for j in range(NSUB)
                ]
                nb = (b + 1) % NBUF

                @pl.when(c + 1 < nch)
                def _():
                    pltpu.async_copy(
                        idx_hbm.at[pl.ds(off + CHUNK, CHUNK)], idx_v.at[nb], isem[nb]
                    )

                for cp in copies:
                    cp.wait()
                pltpu.async_copy(rows_v.at[b], out_hbm.at[pl.ds(off, CHUNK)], osem[b])
            return carry

        lax.fori_loop(0, nch // NBUF, step, 0)
        for b in range(NBUF):
            pltpu.make_async_copy(
                rows_v.at[b], out_hbm.at[pl.ds(base, CHUNK)], osem[b]
            ).wait()

    return body(idx_flat, table)


def kernel(input, W):
    idx = input.reshape(-1).astype(jnp.int32)
    bpw = idx.shape[0] // NW
    out = _emb_lookup(idx, W, bpw)
    return out.reshape(input.shape + (W.shape[1],))
